# baseline (speedup 1.0000x reference)
import functools

import jax
import jax.numpy as jnp
from jax.experimental import pallas as pl
from jax.experimental.pallas import tpu as pltpu

TOPK = 32
T = 1024
D = 128
M = 65536
M_BLK = 4096
N_BLK = M // M_BLK
N_CHUNK = M_BLK // 256
INT_MAX = 2**31 - 1


def _sims_body(tok_ref, keys_ref, sims_ref, m16_ref, m2_ref):
    i = pl.program_id(0)
    tok = tok_ref[...]
    keys = keys_ref[...]
    s = jax.lax.dot_general(tok, keys, (((1,), (1,)), ((), ())),
                            preferred_element_type=jnp.float32)
    sims_ref[...] = s
    m16 = s[:, 0:256]
    for j in range(1, N_CHUNK):
        m16 = jnp.maximum(m16, s[:, j * 256:(j + 1) * 256])
    m16_ref[...] = m16

    @pl.when(i == 0)
    def _():
        m2_ref[...] = m16

    @pl.when(i > 0)
    def _():
        m2_ref[...] = jnp.maximum(m2_ref[...], m16)


def _sims_stage(tokens, mem_keys):
    return pl.pallas_call(
        _sims_body,
        grid=(N_BLK,),
        in_specs=[
            pl.BlockSpec((T, D), lambda i: (0, 0)),
            pl.BlockSpec((M_BLK, D), lambda i: (i, 0)),
        ],
        out_specs=[
            pl.BlockSpec((T, M_BLK), lambda i: (0, i)),
            pl.BlockSpec((T, 256), lambda i: (0, i)),
            pl.BlockSpec((T, 256), lambda i: (0, 0)),
        ],
        out_shape=[
            jax.ShapeDtypeStruct((T, M), jnp.float32),
            jax.ShapeDtypeStruct((T, N_BLK * 256), jnp.float32),
            jax.ShapeDtypeStruct((T, 256), jnp.float32),
        ],
    )(tokens, mem_keys)


def _extract_top32(xs_ref, aux, vals_s, sel_s):
    n = aux.shape[1]
    lane32 = jax.lax.broadcasted_iota(jnp.int32, (T, TOPK), 1)

    def step(k, _):
        x = xs_ref[...]
        m = jnp.max(x, axis=1, keepdims=True)
        eq = x == m
        cand = jnp.min(jnp.where(eq, aux, INT_MAX), axis=1, keepdims=True)
        vals_s[...] = jnp.where(lane32 == k, m, vals_s[...])
        sel_s[...] = jnp.where(lane32 == k, cand, sel_s[...])
        xs_ref[...] = jnp.where(eq & (aux == cand), -jnp.inf, x)
        return 0

    jax.lax.fori_loop(0, TOPK, step, 0, unroll=False)


def _expand16(sel32f):
    rj = jax.lax.broadcasted_iota(jnp.int32, (TOPK, TOPK * 16), 0)
    cp = jax.lax.broadcasted_iota(jnp.int32, (TOPK, TOPK * 16), 1)
    E = (rj == cp // 16).astype(jnp.float32)
    return jax.lax.dot_general(sel32f, E, (((1,), (0,)), ((), ())),
                               preferred_element_type=jnp.float32)


def _b_body(m2_ref, c1_ref, xs_ref, vals_s, sel_s):
    xs_ref[...] = m2_ref[...]
    aux = jax.lax.broadcasted_iota(jnp.int32, (T, 256), 1)
    _extract_top32(xs_ref, aux, vals_s, sel_s)
    l_exp = _expand16(sel_s[...].astype(jnp.float32)).astype(jnp.int32)
    row = jax.lax.broadcasted_iota(jnp.int32, (T, 512), 0)
    col = jax.lax.broadcasted_iota(jnp.int32, (T, 512), 1)
    c1_ref[...] = row * M_BLK + (col % 16) * 256 + l_exp


def _b_stage(m2):
    return pl.pallas_call(
        _b_body,
        in_specs=[pl.BlockSpec((T, 256), lambda: (0, 0))],
        out_specs=pl.BlockSpec((T, 512), lambda: (0, 0)),
        out_shape=jax.ShapeDtypeStruct((T, 512), jnp.int32),
        scratch_shapes=[pltpu.VMEM((T, 256), jnp.float32),
                        pltpu.VMEM((T, TOPK), jnp.float32),
                        pltpu.VMEM((T, TOPK), jnp.int32)],
    )(m2)


def _d_body(c1v_ref, c1i_ref, e2_ref, xs_ref, vals_s, sel_s):
    xs_ref[...] = c1v_ref[...]
    _extract_top32(xs_ref, c1i_ref[...], vals_s, sel_s)
    sel = sel_s[...] % M_BLK
    b = sel // 256
    l = sel % 256
    base_g = (b * M_BLK + l).astype(jnp.float32)
    g_exp = _expand16(base_g).astype(jnp.int32)
    col = jax.lax.broadcasted_iota(jnp.int32, (T, 512), 1)
    e2_ref[...] = g_exp + (col % 16) * 256


def _d_stage(cand1, c1idx):
    return pl.pallas_call(
        _d_body,
        in_specs=[pl.BlockSpec((T, 512), lambda: (0, 0)),
                  pl.BlockSpec((T, 512), lambda: (0, 0))],
        out_specs=pl.BlockSpec((T, 512), lambda: (0, 0)),
        out_shape=jax.ShapeDtypeStruct((T, 512), jnp.int32),
        scratch_shapes=[pltpu.VMEM((T, 512), jnp.float32),
                        pltpu.VMEM((T, TOPK), jnp.float32),
                        pltpu.VMEM((T, TOPK), jnp.int32)],
    )(cand1, c1idx)


def _f_body(c2v_ref, e2i_ref, wn_ref, vrow_ref, xs_ref, vals_s, sel_s):
    xs_ref[...] = c2v_ref[...]
    _extract_top32(xs_ref, e2i_ref[...], vals_s, sel_s)
    vrow_ref[...] = sel_s[...]
    w = vals_s[...]
    mx = jnp.max(w, axis=1, keepdims=True)
    e = jnp.exp(w - mx)
    wn_ref[...] = e / jnp.sum(e, axis=1, keepdims=True)


def _f_stage(cand2, eidx2):
    return pl.pallas_call(
        _f_body,
        in_specs=[pl.BlockSpec((T, 512), lambda: (0, 0)),
                  pl.BlockSpec((T, 512), lambda: (0, 0))],
        out_specs=[pl.BlockSpec((T, TOPK), lambda: (0, 0)),
                   pl.BlockSpec((T, TOPK), lambda: (0, 0))],
        out_shape=[jax.ShapeDtypeStruct((T, TOPK), jnp.float32),
                   jax.ShapeDtypeStruct((T, TOPK), jnp.int32)],
        scratch_shapes=[pltpu.VMEM((T, 512), jnp.float32),
                        pltpu.VMEM((T, TOPK), jnp.float32),
                        pltpu.VMEM((T, TOPK), jnp.int32)],
    )(cand2, eidx2)


def _mlp_body(rec_ref, wn_ref, tok_ref, w1_ref, b1_ref, g_ref, bb_ref,
              w2_ref, b2_ref, out_ref):
    tb = tok_ref.shape[0]
    r = rec_ref[...].reshape(tb, TOPK, D)
    wn = wn_ref[...]
    ctx = jnp.sum(r * wn[:, :, None], axis=1)
    fused = tok_ref[...] + ctx
    h = jnp.dot(fused, w1_ref[...], preferred_element_type=jnp.float32) + b1_ref[...]
    mu = jnp.mean(h, axis=-1, keepdims=True)
    var = jnp.mean((h - mu) ** 2, axis=-1, keepdims=True)
    h = (h - mu) / jnp.sqrt(var + 1e-5) * g_ref[...] + bb_ref[...]
    h = jnp.maximum(h, 0.0)
    out_ref[...] = jnp.dot(h, w2_ref[...], preferred_element_type=jnp.float32) + b2_ref[...]


def _mlp_stage(recalled, wn, tokens, W1, b1, ln_g, ln_b, W2, b2):
    TB = 256
    nblk = T // TB
    full = lambda i: (0, 0)
    return pl.pallas_call(
        _mlp_body,
        grid=(nblk,),
        in_specs=[
            pl.BlockSpec((TB * TOPK, D), lambda i: (i, 0)),
            pl.BlockSpec((TB, TOPK), lambda i: (i, 0)),
            pl.BlockSpec((TB, D), lambda i: (i, 0)),
            pl.BlockSpec((D, D), full),
            pl.BlockSpec((1, D), full),
            pl.BlockSpec((1, D), full),
            pl.BlockSpec((1, D), full),
            pl.BlockSpec((D, D), full),
            pl.BlockSpec((1, D), full),
        ],
        out_specs=pl.BlockSpec((TB, D), lambda i: (i, 0)),
        out_shape=jax.ShapeDtypeStruct((T, D), jnp.float32),
    )(recalled, wn, tokens, W1, b1.reshape(1, D), ln_g.reshape(1, D),
      ln_b.reshape(1, D), W2, b2.reshape(1, D))


def kernel(input_tensor, mem_keys, mem_values, W1, b1, ln_g, ln_b, W2, b2):
    B, S, _ = input_tensor.shape
    tokens = input_tensor.reshape(T, D)

    sims, m16_all, m2 = _sims_stage(tokens, mem_keys)

    c1idx = _b_stage(m2)
    cand1 = jnp.take(m16_all.reshape(-1), c1idx.reshape(-1),
                     axis=0).reshape(T, 512)
    eidx2 = _d_stage(cand1, c1idx)
    cand2 = jnp.take_along_axis(sims, eidx2, axis=1)
    wn, vrow = _f_stage(cand2, eidx2)
    recalled = jnp.take(mem_values, vrow.reshape(-1), axis=0)

    out = _mlp_stage(recalled, wn, tokens, W1, b1, ln_g, ln_b, W2, b2)
    return out.reshape(B, S, D)


# speedup vs baseline: 3.6576x; 3.6576x over previous
import functools

import jax
import jax.numpy as jnp
from jax.experimental import pallas as pl
from jax.experimental.pallas import tpu as pltpu

TOPK = 32
T = 1024
D = 128
M = 65536
M_BLK = 4096
N_BLK = M // M_BLK
N_CHUNK = M_BLK // 256
INT_MAX = 2**31 - 1


def _sims_body(tok_ref, keys_ref, sims_ref, m16_ref, m2_ref):
    i = pl.program_id(0)
    tok = tok_ref[...]
    keys = keys_ref[...]
    s = jax.lax.dot_general(tok, keys, (((1,), (1,)), ((), ())),
                            preferred_element_type=jnp.float32)
    sims_ref[...] = s
    m16 = s[:, 0:256]
    for j in range(1, N_CHUNK):
        m16 = jnp.maximum(m16, s[:, j * 256:(j + 1) * 256])
    m16_ref[...] = m16

    @pl.when(i == 0)
    def _():
        m2_ref[...] = m16

    @pl.when(i > 0)
    def _():
        m2_ref[...] = jnp.maximum(m2_ref[...], m16)


def _sims_stage(tokens, mem_keys):
    return pl.pallas_call(
        _sims_body,
        grid=(N_BLK,),
        in_specs=[
            pl.BlockSpec((T, D), lambda i: (0, 0)),
            pl.BlockSpec((M_BLK, D), lambda i: (i, 0)),
        ],
        out_specs=[
            pl.BlockSpec((T, M_BLK), lambda i: (0, i)),
            pl.BlockSpec((T, 256), lambda i: (0, i)),
            pl.BlockSpec((T, 256), lambda i: (0, 0)),
        ],
        out_shape=[
            jax.ShapeDtypeStruct((T, M), jnp.float32),
            jax.ShapeDtypeStruct((T, N_BLK * 256), jnp.float32),
            jax.ShapeDtypeStruct((T, 256), jnp.float32),
        ],
    )(tokens, mem_keys)


def _extract_top32(xs_ref, aux, vals_s, sel_s):
    n = aux.shape[1]
    lane32 = jax.lax.broadcasted_iota(jnp.int32, (T, TOPK), 1)

    def step(k, _):
        x = xs_ref[...]
        m = jnp.max(x, axis=1, keepdims=True)
        eq = x == m
        cand = jnp.min(jnp.where(eq, aux, INT_MAX), axis=1, keepdims=True)
        vals_s[...] = jnp.where(lane32 == k, m, vals_s[...])
        sel_s[...] = jnp.where(lane32 == k, cand, sel_s[...])
        xs_ref[...] = jnp.where(eq & (aux == cand), -jnp.inf, x)
        return 0

    jax.lax.fori_loop(0, TOPK, step, 0, unroll=False)


def _expand16(sel32f):
    rj = jax.lax.broadcasted_iota(jnp.int32, (TOPK, TOPK * 16), 0)
    cp = jax.lax.broadcasted_iota(jnp.int32, (TOPK, TOPK * 16), 1)
    E = (rj == cp // 16).astype(jnp.float32)
    return jax.lax.dot_general(sel32f, E, (((1,), (0,)), ((), ())),
                               preferred_element_type=jnp.float32)


def _b_body(m2_ref, c1_ref, xs_ref, vals_s, sel_s):
    xs_ref[...] = m2_ref[...]
    aux = jax.lax.broadcasted_iota(jnp.int32, (T, 256), 1)
    _extract_top32(xs_ref, aux, vals_s, sel_s)
    l_exp = _expand16(sel_s[...].astype(jnp.float32)).astype(jnp.int32)
    row = jax.lax.broadcasted_iota(jnp.int32, (T, 512), 0)
    col = jax.lax.broadcasted_iota(jnp.int32, (T, 512), 1)
    c1_ref[...] = row * M_BLK + (col % 16) * 256 + l_exp


def _b_stage(m2):
    return pl.pallas_call(
        _b_body,
        in_specs=[pl.BlockSpec((T, 256), lambda: (0, 0))],
        out_specs=pl.BlockSpec((T, 512), lambda: (0, 0)),
        out_shape=jax.ShapeDtypeStruct((T, 512), jnp.int32),
        scratch_shapes=[pltpu.VMEM((T, 256), jnp.float32),
                        pltpu.VMEM((T, TOPK), jnp.float32),
                        pltpu.VMEM((T, TOPK), jnp.int32)],
    )(m2)


def _d_body(c1v_ref, c1i_ref, e2_ref, xs_ref, vals_s, sel_s):
    xs_ref[...] = c1v_ref[...]
    _extract_top32(xs_ref, c1i_ref[...], vals_s, sel_s)
    sel = sel_s[...] % M_BLK
    b = sel // 256
    l = sel % 256
    base_g = (b * M_BLK + l).astype(jnp.float32)
    g_exp = _expand16(base_g).astype(jnp.int32)
    col = jax.lax.broadcasted_iota(jnp.int32, (T, 512), 1)
    e2_ref[...] = g_exp + (col % 16) * 256


def _d_stage(cand1, c1idx):
    return pl.pallas_call(
        _d_body,
        in_specs=[pl.BlockSpec((T, 512), lambda: (0, 0)),
                  pl.BlockSpec((T, 512), lambda: (0, 0))],
        out_specs=pl.BlockSpec((T, 512), lambda: (0, 0)),
        out_shape=jax.ShapeDtypeStruct((T, 512), jnp.int32),
        scratch_shapes=[pltpu.VMEM((T, 512), jnp.float32),
                        pltpu.VMEM((T, TOPK), jnp.float32),
                        pltpu.VMEM((T, TOPK), jnp.int32)],
    )(cand1, c1idx)


def _f_body(c2v_ref, e2i_ref, wn_ref, vrow_ref, xs_ref, vals_s, sel_s):
    xs_ref[...] = c2v_ref[...]
    _extract_top32(xs_ref, e2i_ref[...], vals_s, sel_s)
    vrow_ref[...] = sel_s[...]
    w = vals_s[...]
    mx = jnp.max(w, axis=1, keepdims=True)
    e = jnp.exp(w - mx)
    wn_ref[...] = e / jnp.sum(e, axis=1, keepdims=True)


def _f_stage(cand2, eidx2):
    return pl.pallas_call(
        _f_body,
        in_specs=[pl.BlockSpec((T, 512), lambda: (0, 0)),
                  pl.BlockSpec((T, 512), lambda: (0, 0))],
        out_specs=[pl.BlockSpec((T, TOPK), lambda: (0, 0)),
                   pl.BlockSpec((T, TOPK), lambda: (0, 0))],
        out_shape=[jax.ShapeDtypeStruct((T, TOPK), jnp.float32),
                   jax.ShapeDtypeStruct((T, TOPK), jnp.int32)],
        scratch_shapes=[pltpu.VMEM((T, 512), jnp.float32),
                        pltpu.VMEM((T, TOPK), jnp.float32),
                        pltpu.VMEM((T, TOPK), jnp.int32)],
    )(cand2, eidx2)


def _mlp_body(rec_ref, wn_ref, tok_ref, w1_ref, b1_ref, g_ref, bb_ref,
              w2_ref, b2_ref, out_ref):
    tb = tok_ref.shape[0]
    r = rec_ref[...].reshape(tb, TOPK, D)
    wn = wn_ref[...]
    ctx = jnp.sum(r * wn[:, :, None], axis=1)
    fused = tok_ref[...] + ctx
    h = jnp.dot(fused, w1_ref[...], preferred_element_type=jnp.float32) + b1_ref[...]
    mu = jnp.mean(h, axis=-1, keepdims=True)
    var = jnp.mean((h - mu) ** 2, axis=-1, keepdims=True)
    h = (h - mu) / jnp.sqrt(var + 1e-5) * g_ref[...] + bb_ref[...]
    h = jnp.maximum(h, 0.0)
    out_ref[...] = jnp.dot(h, w2_ref[...], preferred_element_type=jnp.float32) + b2_ref[...]


def _mlp_stage(recalled, wn, tokens, W1, b1, ln_g, ln_b, W2, b2):
    TB = 256
    nblk = T // TB
    full = lambda i: (0, 0)
    return pl.pallas_call(
        _mlp_body,
        grid=(nblk,),
        in_specs=[
            pl.BlockSpec((TB * TOPK, D), lambda i: (i, 0)),
            pl.BlockSpec((TB, TOPK), lambda i: (i, 0)),
            pl.BlockSpec((TB, D), lambda i: (i, 0)),
            pl.BlockSpec((D, D), full),
            pl.BlockSpec((1, D), full),
            pl.BlockSpec((1, D), full),
            pl.BlockSpec((1, D), full),
            pl.BlockSpec((D, D), full),
            pl.BlockSpec((1, D), full),
        ],
        out_specs=pl.BlockSpec((TB, D), lambda i: (i, 0)),
        out_shape=jax.ShapeDtypeStruct((T, D), jnp.float32),
    )(recalled, wn, tokens, W1, b1.reshape(1, D), ln_g.reshape(1, D),
      ln_b.reshape(1, D), W2, b2.reshape(1, D))


def kernel(input_tensor, mem_keys, mem_values, W1, b1, ln_g, ln_b, W2, b2):
    B, S, _ = input_tensor.shape
    tokens = input_tensor.reshape(T, D)

    sims, m16_all, m2 = _sims_stage(tokens, mem_keys)

    c1idx = _b_stage(m2)
    cand1 = jnp.take(m16_all.reshape(-1), c1idx.reshape(-1),
                     axis=0, mode="clip").reshape(T, 512)
    eidx2 = _d_stage(cand1, c1idx)
    cand2 = jnp.take_along_axis(sims, eidx2, axis=1,
                                mode="clip")
    wn, vrow = _f_stage(cand2, eidx2)
    recalled = jnp.take(mem_values, vrow.reshape(-1), axis=0,
                        mode="clip")

    out = _mlp_stage(recalled, wn, tokens, W1, b1, ln_g, ln_b, W2, b2)
    return out.reshape(B, S, D)
